# baseline (speedup 1.0000x reference)
import functools

import jax
import jax.numpy as jnp
from jax import lax
from jax.experimental import pallas as pl
from jax.experimental.pallas import tpu as pltpu
from jax.experimental.pallas import tpu_sc as plsc

B = 16384
D = 8
NF = 4
V = 1000
NW = 32
BPW = B // NW
L = 16
ENC_R = NF * D

_SC_MESH = plsc.VectorSubcoreMesh(core_axis_name="c", subcore_axis_name="s")


@functools.partial(
    pl.kernel,
    out_type=jax.ShapeDtypeStruct((ENC_R, B), jnp.float32),
    mesh=_SC_MESH,
    scratch_types=[
        pltpu.VMEM((V * D,), jnp.float32),
        pltpu.VMEM((NF * BPW,), jnp.int32),
        pltpu.VMEM((D, NF * BPW), jnp.float32),
        pltpu.SemaphoreType.DMA,
    ],
    compiler_params=pltpu.CompilerParams(needs_layout_passes=False),
)
def _sc_encode(xc_t, embs, enc_out, tab_v, idx_v, enc_v, sem):
    wid = lax.axis_index("s") * 2 + lax.axis_index("c")
    f = wid % NF
    sbase = (wid // NF) * (NF * BPW)
    copies = [
        pltpu.async_copy(embs.at[pl.ds(f * V * D, V * D)], tab_v, sem),
        pltpu.async_copy(xc_t.at[f, pl.ds(sbase, NF * BPW)], idx_v, sem),
    ]
    for cp in copies:
        cp.wait()

    @pl.loop(0, NF * BPW // L, unroll=8)
    def _(g):
        s = g * L
        idx8 = idx_v[pl.ds(s, L)] * D
        for j in range(D):
            enc_v[j, pl.ds(s, L)] = plsc.load_gather(tab_v, [idx8 + j])

    pltpu.sync_copy(enc_v, enc_out.at[pl.ds(f * D, D), pl.ds(sbase, NF * BPW)])


BM = 4096


def _mlp_body(enc_ref, xt_ref, w1e_ref, w1x_ref, b1_ref, w2_ref, b2_ref,
              w3_ref, b3_ref, o_ref):
    h = jnp.dot(w1e_ref[...], enc_ref[...], preferred_element_type=jnp.float32)
    h = h + jnp.dot(w1x_ref[...], xt_ref[...],
                    preferred_element_type=jnp.float32)
    h = jnp.maximum(h + b1_ref[...], 0.0)
    h = jnp.dot(w2_ref[...], h, preferred_element_type=jnp.float32)
    h = jnp.maximum(h + b2_ref[...], 0.0)
    o_ref[...] = (jnp.dot(w3_ref[...], h, preferred_element_type=jnp.float32)
                  + b3_ref[...])


_mlp = pl.pallas_call(
    _mlp_body,
    grid=(B // BM,),
    in_specs=[
        pl.BlockSpec((ENC_R, BM), lambda i: (0, i)),
        pl.BlockSpec((2, BM), lambda i: (0, i)),
        pl.BlockSpec((40, ENC_R), lambda i: (0, 0)),
        pl.BlockSpec((40, 2), lambda i: (0, 0)),
        pl.BlockSpec((40, 1), lambda i: (0, 0)),
        pl.BlockSpec((40, 40), lambda i: (0, 0)),
        pl.BlockSpec((40, 1), lambda i: (0, 0)),
        pl.BlockSpec((2, 40), lambda i: (0, 0)),
        pl.BlockSpec((2, 1), lambda i: (0, 0)),
    ],
    out_specs=pl.BlockSpec((2, BM), lambda i: (0, i)),
    out_shape=jax.ShapeDtypeStruct((2, B), jnp.float32),
)


def kernel(x, x_classes, emb0, emb1, emb2, emb3, W1, b1, W2, b2, W3, b3):
    xc_t = jnp.transpose(x_classes).astype(jnp.int32)
    embs = jnp.concatenate([emb0.reshape(-1), emb1.reshape(-1),
                            emb2.reshape(-1), emb3.reshape(-1)])
    enc = _sc_encode(xc_t, embs)
    yt = _mlp(enc, x.T, W1[2:].T, W1[:2].T, b1.reshape(-1, 1), W2.T,
              b2.reshape(-1, 1), W3.T, b3.reshape(-1, 1))
    return yt.T


# speedup vs baseline: 1.0339x; 1.0339x over previous
import functools

import jax
import jax.numpy as jnp
from jax import lax
from jax.experimental import pallas as pl
from jax.experimental.pallas import tpu as pltpu
from jax.experimental.pallas import tpu_sc as plsc

B = 16384
D = 8
NF = 4
V = 1000
NW = 32
BPW = B // NW
L = 16
ENC_R = NF * D

_SC_MESH = plsc.VectorSubcoreMesh(core_axis_name="c", subcore_axis_name="s")


@functools.partial(
    pl.kernel,
    out_type=jax.ShapeDtypeStruct((ENC_R, B), jnp.float32),
    mesh=_SC_MESH,
    scratch_types=[
        pltpu.VMEM((V * D,), jnp.float32),
        pltpu.VMEM((NF * BPW,), jnp.int32),
        pltpu.VMEM((D, NF * BPW), jnp.float32),
        pltpu.SemaphoreType.DMA,
    ],
    compiler_params=pltpu.CompilerParams(needs_layout_passes=False),
)
def _sc_encode(xc_t, embs, enc_out, tab_v, idx_v, enc_v, sem):
    wid = lax.axis_index("s") * 2 + lax.axis_index("c")
    f = wid % NF
    sbase = (wid // NF) * (NF * BPW)
    copies = [
        pltpu.async_copy(embs.at[pl.ds(f * V * D, V * D)], tab_v, sem),
        pltpu.async_copy(xc_t.at[f, pl.ds(sbase, NF * BPW)], idx_v, sem),
    ]
    for cp in copies:
        cp.wait()

    half = NF * BPW // 2
    out_cps = []
    for h in range(2):
        @pl.loop(h * (half // L), (h + 1) * (half // L), unroll=8)
        def _(g):
            s = g * L
            idx8 = idx_v[pl.ds(s, L)] * D
            for j in range(D):
                enc_v[j, pl.ds(s, L)] = plsc.load_gather(tab_v, [idx8 + j])

        out_cps.append(pltpu.async_copy(
            enc_v.at[:, pl.ds(h * half, half)],
            enc_out.at[pl.ds(f * D, D), pl.ds(sbase + h * half, half)], sem))
    for cp in out_cps:
        cp.wait()


BM = 8192


def _mlp_body(enc_ref, xt_ref, w1e_ref, w1x_ref, b1_ref, w2_ref, b2_ref,
              w3_ref, b3_ref, o_ref):
    h = jnp.dot(w1e_ref[...], enc_ref[...], preferred_element_type=jnp.float32)
    h = h + jnp.dot(w1x_ref[...], xt_ref[...],
                    preferred_element_type=jnp.float32)
    h = jnp.maximum(h + b1_ref[...], 0.0)
    h = jnp.dot(w2_ref[...], h, preferred_element_type=jnp.float32)
    h = jnp.maximum(h + b2_ref[...], 0.0)
    o_ref[...] = (jnp.dot(w3_ref[...], h, preferred_element_type=jnp.float32)
                  + b3_ref[...])


_mlp = pl.pallas_call(
    _mlp_body,
    grid=(B // BM,),
    in_specs=[
        pl.BlockSpec((ENC_R, BM), lambda i: (0, i)),
        pl.BlockSpec((2, BM), lambda i: (0, i)),
        pl.BlockSpec((40, ENC_R), lambda i: (0, 0)),
        pl.BlockSpec((40, 2), lambda i: (0, 0)),
        pl.BlockSpec((40, 1), lambda i: (0, 0)),
        pl.BlockSpec((40, 40), lambda i: (0, 0)),
        pl.BlockSpec((40, 1), lambda i: (0, 0)),
        pl.BlockSpec((2, 40), lambda i: (0, 0)),
        pl.BlockSpec((2, 1), lambda i: (0, 0)),
    ],
    out_specs=pl.BlockSpec((2, BM), lambda i: (0, i)),
    out_shape=jax.ShapeDtypeStruct((2, B), jnp.float32),
)


def kernel(x, x_classes, emb0, emb1, emb2, emb3, W1, b1, W2, b2, W3, b3):
    xc_t = jnp.transpose(x_classes).astype(jnp.int32)
    embs = jnp.concatenate([emb0.reshape(-1), emb1.reshape(-1),
                            emb2.reshape(-1), emb3.reshape(-1)])
    enc = _sc_encode(xc_t, embs)
    yt = _mlp(enc, x.T, W1[2:].T, W1[:2].T, b1.reshape(-1, 1), W2.T,
              b2.reshape(-1, 1), W3.T, b3.reshape(-1, 1))
    return yt.T
